# baseline (speedup 1.0000x reference)
import functools

import jax
import jax.numpy as jnp
from jax import lax
from jax.experimental import pallas as pl
from jax.experimental.pallas import tpu as pltpu
from jax.experimental.pallas import tpu_sc as plsc

D = 64
B = 4096 * 200
NW = 32
CH = 128
NCH = B // (NW * CH)

_mesh = plsc.VectorSubcoreMesh(core_axis_name="c", subcore_axis_name="s")


@functools.partial(
    pl.kernel,
    mesh=_mesh,
    compiler_params=pltpu.CompilerParams(use_tc_tiling_on_sc=False),
    out_type=jax.ShapeDtypeStruct((B, D), jnp.float32),
    scratch_types=[
        pltpu.VMEM((NCH, CH), jnp.int32),
        pltpu.VMEM((CH, D), jnp.float32),
        pltpu.SemaphoreType.DMA,
    ],
)
def _emb_lookup(idx_hbm, table_hbm, out_hbm, idx_v, rows_v, sem):
    wid = lax.axis_index("s") * 2 + lax.axis_index("c")
    base = wid * NCH
    pltpu.sync_copy(idx_hbm.at[pl.ds(base, NCH)], idx_v)

    def body(c, carry):
        pltpu.async_copy(table_hbm.at[idx_v.at[c]], rows_v, sem).wait()
        pltpu.sync_copy(rows_v, out_hbm.at[pl.ds((base + c) * CH, CH)])
        return carry

    lax.fori_loop(0, NCH, body, 0)


def kernel(x, emb):
    idx = x.reshape(B // CH, CH).astype(jnp.int32)
    out = _emb_lookup(idx, emb)
    return out.reshape(x.shape[0], x.shape[1], D)


# speedup vs baseline: 1.1129x; 1.1129x over previous
import functools

import jax
import jax.numpy as jnp
from jax import lax
from jax.experimental import pallas as pl
from jax.experimental.pallas import tpu as pltpu
from jax.experimental.pallas import tpu_sc as plsc

D = 64
B = 4096 * 200
NW = 32
CH = 128
NCH = B // (NW * CH)
K = 4
GCH = K * CH
NG = NCH // K
NGP = NG // 2

_mesh = plsc.VectorSubcoreMesh(core_axis_name="c", subcore_axis_name="s")


@functools.partial(
    pl.kernel,
    mesh=_mesh,
    compiler_params=pltpu.CompilerParams(use_tc_tiling_on_sc=False),
    out_type=jax.ShapeDtypeStruct((B, D), jnp.float32),
    scratch_types=[
        pltpu.VMEM((NCH, CH), jnp.int32),
        pltpu.VMEM((2, GCH, D), jnp.float32),
        pltpu.SemaphoreType.DMA,
        pltpu.SemaphoreType.DMA,
    ],
)
def _emb_lookup(idx_hbm, table_hbm, out_hbm, idx_v, rows_v, in_sem, out_sem):
    wid = lax.axis_index("s") * 2 + lax.axis_index("c")
    chunk0 = wid * NCH
    pltpu.sync_copy(idx_hbm.at[pl.ds(chunk0, NCH)], idx_v)

    def group(g, db):
        @pl.when(g >= 2)
        def _():
            pltpu.make_async_copy(
                out_hbm.at[pl.ds(0, GCH)], rows_v.at[db], out_sem
            ).wait()

        descs = [
            pltpu.async_copy(
                table_hbm.at[idx_v.at[g * K + j]],
                rows_v.at[db, pl.ds(j * CH, CH)],
                in_sem,
            )
            for j in range(K)
        ]
        for d in descs:
            d.wait()
        pltpu.async_copy(
            rows_v.at[db],
            out_hbm.at[pl.ds((chunk0 + g * K) * CH, GCH)],
            out_sem,
        )

    def body(gp, carry):
        group(gp * 2, 0)
        group(gp * 2 + 1, 1)
        return carry

    lax.fori_loop(0, NGP, body, 0)
    for db in range(2):
        pltpu.make_async_copy(
            out_hbm.at[pl.ds(0, GCH)], rows_v.at[db], out_sem
        ).wait()


def kernel(x, emb):
    idx = x.reshape(B // CH, CH).astype(jnp.int32)
    out = _emb_lookup(idx, emb)
    return out.reshape(x.shape[0], x.shape[1], D)
